# baseline (speedup 1.0000x reference)
import functools

import jax
import jax.numpy as jnp
from jax.experimental import pallas as pl
from jax.experimental.pallas import tpu as pltpu

_NEG_POS_RATIO = 3
_ROWS = 8
_CP = 512


def _softplus(s):
    return jnp.maximum(s, 0.0) + jnp.log1p(jnp.exp(-jnp.abs(s)))


def _conf_head(conf_ref, m2e_ref, acc_ref, islot, cslot, sacc_ref,
               num_priors, evenf):
    w2 = 2 * num_priors
    cw2 = 2 * _CP
    npr = jnp.zeros((_ROWS, 1), jnp.float32)
    for c0 in range(0, w2, cw2):
        cw = min(cw2, w2 - c0)
        c = conf_ref[:, c0:c0 + cw]
        s = jnp.roll(c, -1, axis=1) - c
        m2e = m2e_ref[:, c0:c0 + cw]
        loss = _softplus(s)
        acc_ref[:, 0:cw] += loss * evenf[:, 0:cw] - m2e * s
        npr += jnp.sum(m2e, axis=1, keepdims=True)

    npc = jnp.maximum(npr, 1.0)
    sacc_ref[islot] += jnp.sum(npc)
    kv = jnp.float32(_NEG_POS_RATIO) * npc
    cnt_neg = jnp.float32(num_priors) - npr
    rowflag = kv < cnt_neg

    @pl.when(jnp.any(rowflag))
    def _topk_correction():
        c = conf_ref[...]
        s = jnp.roll(c, -1, axis=1) - c
        m2e = m2e_ref[...]
        lane = jax.lax.broadcasted_iota(jnp.int32, c.shape, 1)
        valid = ((lane & 1) == 0) & (m2e == 0.0)
        loss = _softplus(s)
        bits = jax.lax.bitcast_convert_type(s, jnp.uint32)
        sign = bits >> jnp.uint32(31)
        flip = jnp.where(sign == jnp.uint32(1),
                         jnp.uint32(0xFFFFFFFF), jnp.uint32(0x80000000))
        ukey = bits ^ flip

        def body(i, cur):
            bit = jnp.uint32(31) - i.astype(jnp.uint32)
            test = cur | (jnp.uint32(1) << bit)
            cnt = jnp.sum(jnp.where(valid & (ukey >= test), 1.0, 0.0),
                          axis=1, keepdims=True)
            return jnp.where(cnt >= kv, test, cur)

        t_key = jax.lax.fori_loop(
            0, 32, body, jnp.zeros((_ROWS, 1), jnp.uint32))
        gtm = valid & (ukey > t_key)
        n_gt = jnp.sum(jnp.where(gtm, 1.0, 0.0), axis=1, keepdims=True)
        t_bits = jnp.where(t_key >= jnp.uint32(0x80000000),
                           t_key ^ jnp.uint32(0x80000000), ~t_key)
        t_val = jax.lax.bitcast_convert_type(t_bits, jnp.float32)
        gt_sum = jnp.sum(jnp.where(gtm, loss, 0.0), axis=1, keepdims=True)
        topk_row = gt_sum + (kv - n_gt) * _softplus(t_val)
        neg_all_row = jnp.sum(jnp.where(valid, loss, 0.0),
                              axis=1, keepdims=True)
        sacc_ref[cslot] += jnp.sum(
            jnp.where(rowflag, topk_row - neg_all_row, 0.0))


def _ssd_kernel(loc_ref, loct_ref, mask4_ref, conf_p_ref, m2e_p_ref,
                conf_b_ref, m2e_b_ref,
                out_l_ref, out_pc_ref, out_bc_ref,
                accl_ref, accp_ref, accb_ref, sacc_ref,
                *, num_priors, num_steps):
    g = pl.program_id(0)

    @pl.when(g == 0)
    def _init():
        accl_ref[...] = jnp.zeros_like(accl_ref)
        accp_ref[...] = jnp.zeros_like(accp_ref)
        accb_ref[...] = jnp.zeros_like(accb_ref)
        for i in range(4):
            sacc_ref[i] = 0.0

    w4 = 4 * num_priors
    cw4 = 4 * _CP
    for c0 in range(0, w4, cw4):
        cw = min(cw4, w4 - c0)
        d = loc_ref[:, c0:c0 + cw] - loct_ref[:, c0:c0 + cw]
        a = jnp.abs(d)
        l1 = jnp.where(a < 1.0, 0.5 * a * a, a - 0.5)
        accl_ref[:, 0:cw] += l1 * mask4_ref[:, c0:c0 + cw]

    lane = jax.lax.broadcasted_iota(jnp.int32, (_ROWS, 2 * _CP), 1)
    evenf = ((lane & 1) == 0).astype(jnp.float32)
    _conf_head(conf_p_ref, m2e_p_ref, accp_ref, 0, 2, sacc_ref,
               num_priors, evenf)
    _conf_head(conf_b_ref, m2e_b_ref, accb_ref, 1, 3, sacc_ref,
               num_priors, evenf)

    @pl.when(g == num_steps - 1)
    def _finish():
        np_p = sacc_ref[0]
        l_tot = jnp.sum(accl_ref[...])
        pc_tot = jnp.sum(accp_ref[...]) + sacc_ref[2]
        bc_tot = jnp.sum(accb_ref[...]) + sacc_ref[3]
        out_l_ref[...] = jnp.broadcast_to(l_tot / np_p, (1, 1))
        out_pc_ref[...] = jnp.broadcast_to(pc_tot / np_p, (1, 1))
        out_bc_ref[...] = jnp.broadcast_to(bc_tot / sacc_ref[1], (1, 1))


def kernel(player_loc, player_conf, ball_conf, player_loc_t, player_conf_t,
           ball_conf_t):
    B = player_loc.shape[0]
    player_loc = player_loc.reshape(B, -1, 4)
    P = player_loc.shape[1]

    loc = player_loc.reshape(B, 4 * P)
    loct = player_loc_t.reshape(B, 4 * P)
    posf_p = (player_conf_t.reshape(B, P) > 0).astype(jnp.float32)
    posf_b = (ball_conf_t.reshape(B, P) > 0).astype(jnp.float32)
    mask4 = jnp.broadcast_to(posf_p[:, :, None], (B, P, 4)).reshape(B, 4 * P)
    ev = jnp.array([1.0, 0.0], jnp.float32)
    m2e_p = (posf_p[:, :, None] * ev).reshape(B, 2 * P)
    m2e_b = (posf_b[:, :, None] * ev).reshape(B, 2 * P)
    conf_p = player_conf.reshape(B, 2 * P)
    conf_b = ball_conf.reshape(B, 2 * P)

    num_steps = B // _ROWS
    spec4 = pl.BlockSpec((_ROWS, 4 * P), lambda i: (i, 0))
    spec2 = pl.BlockSpec((_ROWS, 2 * P), lambda i: (i, 0))
    out_spec = pl.BlockSpec((1, 1), lambda i: (0, 0))
    out_ty = jax.ShapeDtypeStruct((1, 1), jnp.float32)

    out_l, out_pc, out_bc = pl.pallas_call(
        functools.partial(_ssd_kernel, num_priors=P, num_steps=num_steps),
        grid=(num_steps,),
        in_specs=[spec4, spec4, spec4, spec2, spec2, spec2, spec2],
        out_specs=[out_spec, out_spec, out_spec],
        out_shape=[out_ty, out_ty, out_ty],
        scratch_shapes=[pltpu.VMEM((_ROWS, 4 * _CP), jnp.float32),
                        pltpu.VMEM((_ROWS, 2 * _CP), jnp.float32),
                        pltpu.VMEM((_ROWS, 2 * _CP), jnp.float32),
                        pltpu.SMEM((4,), jnp.float32)],
    )(loc, loct, mask4, conf_p, m2e_p, conf_b, m2e_b)

    return (out_l[0, 0], out_pc[0, 0], out_bc[0, 0])


# speedup vs baseline: 1.6341x; 1.6341x over previous
import functools

import jax
import jax.numpy as jnp
from jax.experimental import pallas as pl
from jax.experimental.pallas import tpu as pltpu

_NEG_POS_RATIO = 3
_ROWS = 8
_C_CONF = 1024
_C_LOC = 2048


def _softplus_pair(s):
    ell = jnp.log1p(jnp.exp(-jnp.abs(s)))
    return jnp.maximum(s, 0.0) + ell, jnp.maximum(-s, 0.0) + ell


def _ssd_kernel(loc_ref, loct_ref, mask4_ref,
                s_p_ref, lab_p_ref, s_b_ref, lab_b_ref,
                out_l_ref, out_pc_ref, out_bc_ref,
                accl_ref, accp_ref, accb_ref, sacc_ref,
                *, num_priors, num_steps):
    g = pl.program_id(0)

    @pl.when(g == 0)
    def _init():
        accl_ref[...] = jnp.zeros_like(accl_ref)
        accp_ref[...] = jnp.zeros_like(accp_ref)
        accb_ref[...] = jnp.zeros_like(accb_ref)
        for i in range(4):
            sacc_ref[i] = 0.0

    w4 = 4 * num_priors
    for c0 in range(0, w4, _C_LOC):
        cw = min(_C_LOC, w4 - c0)
        d = loc_ref[:, c0:c0 + cw] - loct_ref[:, c0:c0 + cw]
        a = jnp.abs(d)
        l1 = jnp.where(a < 1.0, 0.5 * a * a, a - 0.5)
        accl_ref[:, 0:cw] += l1 * mask4_ref[:, c0:c0 + cw]

    for s_ref, lab_ref, acc_ref, islot, cslot in (
            (s_p_ref, lab_p_ref, accp_ref, 0, 2),
            (s_b_ref, lab_b_ref, accb_ref, 1, 3)):
        npr = jnp.zeros((_ROWS, 1), jnp.float32)
        for c0 in range(0, num_priors, _C_CONF):
            cw = min(_C_CONF, num_priors - c0)
            s = s_ref[:, c0:c0 + cw]
            pos = lab_ref[:, c0:c0 + cw] > 0
            loss, ce_pos = _softplus_pair(s)
            acc_ref[:, 0:cw] += jnp.where(pos, ce_pos, loss)
            npr += jnp.sum(pos.astype(jnp.float32), axis=1, keepdims=True)

        npc = jnp.maximum(npr, 1.0)
        sacc_ref[islot] += jnp.sum(npc)
        kv = jnp.float32(_NEG_POS_RATIO) * npc
        cnt_neg = jnp.float32(num_priors) - npr
        rowflag = kv < cnt_neg

        @pl.when(jnp.any(rowflag))
        def _topk_correction(s_ref=s_ref, lab_ref=lab_ref,
                             rowflag=rowflag, kv=kv, cslot=cslot):
            s = s_ref[...]
            pos = lab_ref[...] > 0
            negm = jnp.logical_not(pos)
            loss, _ = _softplus_pair(s)
            bits = jax.lax.bitcast_convert_type(s, jnp.uint32)
            sign = bits >> jnp.uint32(31)
            flip = jnp.where(sign == jnp.uint32(1),
                             jnp.uint32(0xFFFFFFFF), jnp.uint32(0x80000000))
            ukey = bits ^ flip

            def body(i, cur):
                bit = jnp.uint32(31) - i.astype(jnp.uint32)
                test = cur | (jnp.uint32(1) << bit)
                cnt = jnp.sum(jnp.where(negm & (ukey >= test), 1.0, 0.0),
                              axis=1, keepdims=True)
                return jnp.where(cnt >= kv, test, cur)

            t_key = jax.lax.fori_loop(
                0, 32, body, jnp.zeros((_ROWS, 1), jnp.uint32))
            gtm = negm & (ukey > t_key)
            n_gt = jnp.sum(jnp.where(gtm, 1.0, 0.0), axis=1, keepdims=True)
            t_bits = jnp.where(t_key >= jnp.uint32(0x80000000),
                               t_key ^ jnp.uint32(0x80000000), ~t_key)
            t_val = jax.lax.bitcast_convert_type(t_bits, jnp.float32)
            t_loss, _ = _softplus_pair(t_val)
            gt_sum = jnp.sum(jnp.where(gtm, loss, 0.0), axis=1, keepdims=True)
            topk_row = gt_sum + (kv - n_gt) * t_loss
            neg_all_row = jnp.sum(jnp.where(pos, 0.0, loss),
                                  axis=1, keepdims=True)
            sacc_ref[cslot] += jnp.sum(
                jnp.where(rowflag, topk_row - neg_all_row, 0.0))

    @pl.when(g == num_steps - 1)
    def _finish():
        np_p = sacc_ref[0]
        l_tot = jnp.sum(accl_ref[...])
        pc_tot = jnp.sum(accp_ref[...]) + sacc_ref[2]
        bc_tot = jnp.sum(accb_ref[...]) + sacc_ref[3]
        out_l_ref[...] = jnp.broadcast_to(l_tot / np_p, (1, 1))
        out_pc_ref[...] = jnp.broadcast_to(pc_tot / np_p, (1, 1))
        out_bc_ref[...] = jnp.broadcast_to(bc_tot / sacc_ref[1], (1, 1))


def kernel(player_loc, player_conf, ball_conf, player_loc_t, player_conf_t,
           ball_conf_t):
    B = player_loc.shape[0]
    player_loc = player_loc.reshape(B, -1, 4)
    P = player_loc.shape[1]

    loc = player_loc.reshape(B, 4 * P)
    loct = player_loc_t.reshape(B, 4 * P)
    lab_p = player_conf_t.reshape(B, P).astype(jnp.int32)
    lab_b = ball_conf_t.reshape(B, P).astype(jnp.int32)
    mask4 = jnp.broadcast_to((lab_p > 0)[:, :, None].astype(jnp.float32),
                             (B, P, 4)).reshape(B, 4 * P)
    cvec = jnp.array([-1.0, 1.0], jnp.float32)
    s_p = jnp.sum(player_conf.reshape(B, P, 2) * cvec, axis=-1)
    s_b = jnp.sum(ball_conf.reshape(B, P, 2) * cvec, axis=-1)

    num_steps = B // _ROWS
    spec4 = pl.BlockSpec((_ROWS, 4 * P), lambda i: (i, 0))
    spec1 = pl.BlockSpec((_ROWS, P), lambda i: (i, 0))
    out_spec = pl.BlockSpec((1, 1), lambda i: (0, 0))
    out_ty = jax.ShapeDtypeStruct((1, 1), jnp.float32)

    out_l, out_pc, out_bc = pl.pallas_call(
        functools.partial(_ssd_kernel, num_priors=P, num_steps=num_steps),
        grid=(num_steps,),
        in_specs=[spec4, spec4, spec4, spec1, spec1, spec1, spec1],
        out_specs=[out_spec, out_spec, out_spec],
        out_shape=[out_ty, out_ty, out_ty],
        scratch_shapes=[pltpu.VMEM((_ROWS, _C_LOC), jnp.float32),
                        pltpu.VMEM((_ROWS, _C_CONF), jnp.float32),
                        pltpu.VMEM((_ROWS, _C_CONF), jnp.float32),
                        pltpu.SMEM((4,), jnp.float32)],
    )(loc, loct, mask4, s_p, lab_p, s_b, lab_b)

    return (out_l[0, 0], out_pc[0, 0], out_bc[0, 0])
